# Initial kernel scaffold; baseline (speedup 1.0000x reference)
import functools

import jax
import jax.numpy as jnp
from jax import lax
from jax.experimental import pallas as pl
from jax.experimental.pallas import tpu as pltpu
from jax.experimental.pallas import tpu_sc as plsc

N = 10000
E = 320000
D = 128
HEADS = 8
HDIM = 64
NBLK = 4
BPAIR = 4096
SA_H = 64
FC_H = 128

NC = 2
NS = 16
NW = NC * NS
CH = 80
CHUNKS = E // (NW * CH)
ROWS_PER_TILE = N // NS
ZR = 125

_MESH = plsc.VectorSubcoreMesh(core_axis_name="c", subcore_axis_name="s")


def _worker_id():
    return lax.axis_index("s") * NC + lax.axis_index("c")


def _zero_fill(zbuf, acc_list):
    w = zbuf.shape[1]

    def zrow(i, _):
        for j in range(w // 16):
            zbuf[i, pl.ds(j * 16, 16)] = jnp.zeros((16,), jnp.float32)
        return 0

    lax.fori_loop(0, ZR, zrow, 0)
    sid = lax.axis_index("s")
    for acc in acc_list:
        for t in range(ROWS_PER_TILE // ZR):
            pltpu.sync_copy(zbuf, acc.at[pl.ds(sid * ROWS_PER_TILE + t * ZR, ZR)])
    plsc.subcore_barrier()


def _dump_acc(acc, out_ref, dump, cid):
    sid = lax.axis_index("s")
    for t in range(ROWS_PER_TILE // ZR):
        r0 = sid * ROWS_PER_TILE + t * ZR
        pltpu.sync_copy(acc.at[pl.ds(r0, ZR)], dump)
        pltpu.sync_copy(dump, out_ref.at[cid, pl.ds(r0, ZR)])


@functools.partial(
    pl.kernel,
    out_type=jax.ShapeDtypeStruct((2, 2, N, 16), jnp.float32),
    mesh=_MESH,
    scratch_types=[
        pltpu.VMEM((CH,), jnp.int32),
        pltpu.VMEM((CH,), jnp.int32),
        pltpu.VMEM((CH, 16), jnp.float32),
        pltpu.VMEM((ZR, 16), jnp.float32),
        pltpu.VMEM_SHARED((N, 16), jnp.float32),
        pltpu.VMEM_SHARED((N, 16), jnp.float32),
    ],
)
def _sc_degrees(src_hbm, dst_hbm, out_hbm, src_v, dst_v, ones_v, zbuf,
                acc_s, acc_d):
    cid = lax.axis_index("c")
    wid = _worker_id()
    _zero_fill(zbuf, [acc_s, acc_d])

    def fill(i, _):
        ones_v[i, :] = jnp.ones((16,), jnp.float32)
        return 0

    lax.fori_loop(0, CH, fill, 0)

    def body(j, _):
        off = (j * NW + wid) * CH
        pltpu.sync_copy(src_hbm.at[pl.ds(off, CH)], src_v)
        pltpu.sync_copy(dst_hbm.at[pl.ds(off, CH)], dst_v)
        pltpu.sync_copy(ones_v, acc_s.at[src_v], add=True)
        pltpu.sync_copy(ones_v, acc_d.at[dst_v], add=True)
        return 0

    lax.fori_loop(0, CHUNKS, body, 0)
    plsc.subcore_barrier()
    _dump_acc(acc_s, out_hbm.at[cid], zbuf, 0)
    _dump_acc(acc_d, out_hbm.at[cid], zbuf, 1)


@functools.partial(
    pl.kernel,
    out_type=jax.ShapeDtypeStruct((2, N, D), jnp.float32),
    mesh=_MESH,
    scratch_types=[
        pltpu.VMEM((CH,), jnp.int32),
        pltpu.VMEM((CH,), jnp.int32),
        pltpu.VMEM((CH, D), jnp.float32),
        pltpu.VMEM((ZR, D), jnp.float32),
        pltpu.VMEM_SHARED((N, D), jnp.float32),
        pltpu.SemaphoreType.DMA,
    ],
)
def _sc_gcn_agg(h_hbm, src_hbm, dst_hbm, out_hbm, src_v, dst_v, rows_v,
                zbuf, acc, sem):
    cid = lax.axis_index("c")
    wid = _worker_id()
    _zero_fill(zbuf, [acc])

    def body(j, _):
        off = (j * NW + wid) * CH
        pltpu.sync_copy(src_hbm.at[pl.ds(off, CH)], src_v)
        pltpu.sync_copy(dst_hbm.at[pl.ds(off, CH)], dst_v)
        pltpu.async_copy(h_hbm.at[src_v], rows_v, sem).wait()
        pltpu.sync_copy(rows_v, acc.at[dst_v], add=True)
        return 0

    lax.fori_loop(0, CHUNKS, body, 0)
    plsc.subcore_barrier()
    _dump_acc(acc, out_hbm, zbuf, cid)


@functools.partial(
    pl.kernel,
    out_type=(
        jax.ShapeDtypeStruct((E, 16), jnp.float32),
        jax.ShapeDtypeStruct((2, N, 16), jnp.float32),
    ),
    mesh=_MESH,
    scratch_types=[
        pltpu.VMEM((CH,), jnp.int32),
        pltpu.VMEM((CH,), jnp.int32),
        pltpu.VMEM((CH, 16), jnp.float32),
        pltpu.VMEM((CH, 16), jnp.float32),
        pltpu.VMEM((CH, 16), jnp.float32),
        pltpu.VMEM((8, 16), jnp.float32),
        pltpu.VMEM((ZR, 16), jnp.float32),
        pltpu.VMEM_SHARED((N, 16), jnp.float32),
        pltpu.SemaphoreType.DMA,
        pltpu.SemaphoreType.DMA,
    ],
)
def _sc_gat_edge(el_hbm, er_hbm, ml_hbm, mr_hbm, src_hbm, dst_hbm,
                 a_hbm, den_hbm, src_v, dst_v, l_v, r_v, a_v, m_v, zbuf,
                 acc, sem_l, sem_r):
    cid = lax.axis_index("c")
    wid = _worker_id()
    _zero_fill(zbuf, [acc])
    pltpu.sync_copy(ml_hbm, m_v.at[pl.ds(0, 1)])
    pltpu.sync_copy(mr_hbm, m_v.at[pl.ds(1, 1)])
    msum = m_v[0, :] + m_v[1, :]
    cvec = jnp.maximum(msum, 0.2 * msum)

    def body(j, _):
        off = (j * NW + wid) * CH
        pltpu.sync_copy(src_hbm.at[pl.ds(off, CH)], src_v)
        pltpu.sync_copy(dst_hbm.at[pl.ds(off, CH)], dst_v)
        cl = pltpu.async_copy(el_hbm.at[src_v], l_v, sem_l)
        cr = pltpu.async_copy(er_hbm.at[dst_v], r_v, sem_r)
        cl.wait()
        cr.wait()

        def edge(k, _):
            v = l_v[k, :] + r_v[k, :]
            e = jnp.maximum(v, 0.2 * v)
            a_v[k, :] = jnp.exp(e - cvec)
            return 0

        lax.fori_loop(0, CH, edge, 0)
        pltpu.sync_copy(a_v, acc.at[dst_v], add=True)
        pltpu.sync_copy(a_v, a_hbm.at[pl.ds(off, CH)])
        return 0

    lax.fori_loop(0, CHUNKS, body, 0)
    plsc.subcore_barrier()
    _dump_acc(acc, den_hbm, zbuf, cid)


def _make_sc_gat_agg(h0):
    @functools.partial(
        pl.kernel,
        out_type=jax.ShapeDtypeStruct((2, N, D), jnp.float32),
        mesh=_MESH,
        scratch_types=[
            pltpu.VMEM((CH,), jnp.int32),
            pltpu.VMEM((CH,), jnp.int32),
            pltpu.VMEM((CH, D), jnp.float32),
            pltpu.VMEM((CH, 16), jnp.float32),
            pltpu.VMEM((CH, 16), jnp.float32),
            pltpu.VMEM((ZR, D), jnp.float32),
            pltpu.VMEM_SHARED((N, D), jnp.float32),
            pltpu.SemaphoreType.DMA,
            pltpu.SemaphoreType.DMA,
        ],
    )
    def _sc_gat_agg(feat_hbm, a_hbm, den_hbm, src_hbm, dst_hbm, out_hbm,
                    src_v, dst_v, rows_v, a_v, den_v, zbuf, acc, sem, sem2):
        cid = lax.axis_index("c")
        wid = _worker_id()
        _zero_fill(zbuf, [acc])

        def body(j, _):
            off = (j * NW + wid) * CH
            pltpu.sync_copy(src_hbm.at[pl.ds(off, CH)], src_v)
            pltpu.sync_copy(dst_hbm.at[pl.ds(off, CH)], dst_v)
            pltpu.sync_copy(a_hbm.at[pl.ds(off, CH)], a_v)
            cf = pltpu.async_copy(feat_hbm.at[src_v], rows_v, sem)
            cd = pltpu.async_copy(den_hbm.at[dst_v], den_v, sem2)
            cf.wait()
            cd.wait()

            def edge(k, _):
                al0 = a_v[k, h0] / den_v[k, h0]
                al1 = a_v[k, h0 + 1] / den_v[k, h0 + 1]
                for q in range(4):
                    sl = pl.ds(q * 16, 16)
                    rows_v[k, sl] = rows_v[k, sl] * al0
                for q in range(4, 8):
                    sl = pl.ds(q * 16, 16)
                    rows_v[k, sl] = rows_v[k, sl] * al1
                return 0

            lax.fori_loop(0, CH, edge, 0)
            pltpu.sync_copy(rows_v, acc.at[dst_v], add=True)
            return 0

        lax.fori_loop(0, CHUNKS, body, 0)
        plsc.subcore_barrier()
        _dump_acc(acc, out_hbm, zbuf, cid)

    return _sc_gat_agg


_SC_GAT_AGG = [_make_sc_gat_agg(2 * s) for s in range(4)]


PB = BPAIR // NW


@functools.partial(
    pl.kernel,
    out_type=jax.ShapeDtypeStruct((3, BPAIR, D), jnp.float32),
    mesh=_MESH,
    scratch_types=[
        pltpu.VMEM((PB,), jnp.int32),
        pltpu.VMEM((PB,), jnp.int32),
        pltpu.VMEM((PB, D), jnp.float32),
        pltpu.SemaphoreType.DMA,
    ],
)
def _sc_pair_gather(g1_hbm, g2_hbm, g3_hbm, p0_hbm, p1_hbm, out_hbm,
                    i0_v, i1_v, rows_v, sem):
    wid = _worker_id()
    base = wid * PB
    pltpu.sync_copy(p0_hbm.at[pl.ds(base, PB)], i0_v)
    pltpu.sync_copy(p1_hbm.at[pl.ds(base, PB)], i1_v)
    pltpu.async_copy(g1_hbm.at[i0_v], rows_v, sem).wait()
    pltpu.sync_copy(rows_v, out_hbm.at[0, pl.ds(base, PB)])
    pltpu.async_copy(g2_hbm.at[i0_v], rows_v, sem).wait()
    pltpu.sync_copy(rows_v, out_hbm.at[1, pl.ds(base, PB)])
    pltpu.async_copy(g3_hbm.at[i1_v], rows_v, sem).wait()
    pltpu.sync_copy(rows_v, out_hbm.at[2, pl.ds(base, PB)])


RB = 1000
GRID = N // RB


def _row_spec(w):
    return pl.BlockSpec((RB, w), lambda i: (i, 0))


def _full_spec(shape):
    return pl.BlockSpec(shape, lambda i: tuple(0 for _ in shape))


def _tc_gcn_pre_body(h_ref, da_ref, db_ref, w_ref, o_ref):
    deg = da_ref[:, 0:1] + db_ref[:, 0:1]
    ns = jnp.where(deg > 0, lax.rsqrt(jnp.maximum(deg, 1e-9)), 0.0)
    o_ref[...] = lax.dot_general(h_ref[...] * ns, w_ref[...],
                                 (((1,), (1,)), ((), ())),
                                 preferred_element_type=jnp.float32)


def _tc_gcn_pre(h, dega, degb, w):
    return pl.pallas_call(
        _tc_gcn_pre_body,
        grid=(GRID,),
        in_specs=[_row_spec(D), _row_spec(16), _row_spec(16),
                  _full_spec((D, D))],
        out_specs=_row_spec(D),
        out_shape=jax.ShapeDtypeStruct((N, D), jnp.float32),
    )(h, dega, degb, w)


def _tc_gcn_post_body(aa_ref, ab_ref, da_ref, db_ref, b_ref, wih_ref,
                      bih_ref, bhh_ref, o_ref):
    deg = da_ref[:, 0:1] + db_ref[:, 0:1]
    nd = jnp.where(deg > 0, lax.rsqrt(jnp.maximum(deg, 1e-9)), 0.0)
    x = (aa_ref[...] + ab_ref[...]) * nd + b_ref[...]
    gi = lax.dot_general(x, wih_ref[...], (((1,), (1,)), ((), ())),
                         preferred_element_type=jnp.float32) + bih_ref[...]
    bhh = bhh_ref[...]
    r = jax.nn.sigmoid(gi[:, 0:D] + bhh[:, 0:D])
    z = jax.nn.sigmoid(gi[:, D:2 * D] + bhh[:, D:2 * D])
    n = jnp.tanh(gi[:, 2 * D:3 * D] + r * bhh[:, 2 * D:3 * D])
    o_ref[...] = jnp.maximum((1.0 - z) * n, 0.0)


def _tc_gcn_post(aggs, dega, degb, b, wih, bih, bhh):
    return pl.pallas_call(
        _tc_gcn_post_body,
        grid=(GRID,),
        in_specs=[_row_spec(D), _row_spec(D), _row_spec(16), _row_spec(16),
                  _full_spec((1, D)), _full_spec((3 * D, D)),
                  _full_spec((1, 3 * D)), _full_spec((1, 3 * D))],
        out_specs=_row_spec(D),
        out_shape=jax.ShapeDtypeStruct((N, D), jnp.float32),
    )(aggs[0], aggs[1], dega, degb, b, wih, bih, bhh)


def _tc_gat_proj_body(f_ref, w_ref, al_ref, ar_ref, feat_ref, el_ref,
                      er_ref, ml_ref, mr_ref):
    feat = lax.dot_general(f_ref[...], w_ref[...], (((1,), (1,)), ((), ())),
                           preferred_element_type=jnp.float32)
    feat_ref[...] = feat
    el = jnp.dot(feat, al_ref[...], preferred_element_type=jnp.float32)
    er = jnp.dot(feat, ar_ref[...], preferred_element_type=jnp.float32)
    el_ref[...] = el
    er_ref[...] = er
    ml = jnp.max(el, axis=0, keepdims=True)
    mr = jnp.max(er, axis=0, keepdims=True)

    @pl.when(pl.program_id(0) == 0)
    def _():
        ml_ref[...] = ml
        mr_ref[...] = mr

    @pl.when(pl.program_id(0) > 0)
    def _():
        ml_ref[...] = jnp.maximum(ml_ref[...], ml)
        mr_ref[...] = jnp.maximum(mr_ref[...], mr)


def _tc_gat_proj(f2, w, al16, ar16):
    return pl.pallas_call(
        _tc_gat_proj_body,
        grid=(GRID,),
        in_specs=[_row_spec(D), _full_spec((HEADS * HDIM, D)),
                  _full_spec((HEADS * HDIM, 16)),
                  _full_spec((HEADS * HDIM, 16))],
        out_specs=[_row_spec(HEADS * HDIM), _row_spec(16), _row_spec(16),
                   _full_spec((1, 16)), _full_spec((1, 16))],
        out_shape=[
            jax.ShapeDtypeStruct((N, HEADS * HDIM), jnp.float32),
            jax.ShapeDtypeStruct((N, 16), jnp.float32),
            jax.ShapeDtypeStruct((N, 16), jnp.float32),
            jax.ShapeDtypeStruct((1, 16), jnp.float32),
            jax.ShapeDtypeStruct((1, 16), jnp.float32),
        ],
    )(f2, w, al16, ar16)


def _tc_add16_body(a_ref, b_ref, o_ref):
    o_ref[...] = a_ref[...] + b_ref[...]


def _tc_add16(a, b):
    return pl.pallas_call(
        _tc_add16_body,
        grid=(GRID,),
        in_specs=[_row_spec(16), _row_spec(16)],
        out_specs=_row_spec(16),
        out_shape=jax.ShapeDtypeStruct((N, 16), jnp.float32),
    )(a, b)


def _tc_sem_body(g0a_ref, g0b_ref, g1a_ref, g1b_ref, b0_ref, b1_ref,
                 w1_ref, sb1_ref, w2_ref, h_ref, w1a_ref, w1b_ref, w1c_ref,
                 o1_ref, o2_ref, o3_ref):
    def elu(x):
        return jnp.where(x > 0, x, jnp.exp(jnp.minimum(x, 0.0)) - 1.0)

    sem0 = elu(g0a_ref[...] + g0b_ref[...] + b0_ref[...])
    sem1 = elu(g1a_ref[...] + g1b_ref[...] + b1_ref[...])

    def score(s):
        t = jnp.tanh(lax.dot_general(s, w1_ref[...], (((1,), (1,)), ((), ())),
                                     preferred_element_type=jnp.float32)
                     + sb1_ref[...])
        return jnp.sum(t * w2_ref[...], axis=1, keepdims=True)

    w0 = score(sem0)
    w1 = score(sem1)
    m = jnp.maximum(w0, w1)
    e0 = jnp.exp(w0 - m)
    e1 = jnp.exp(w1 - m)
    beta0 = e0 / (e0 + e1)
    ui = beta0 * sem0 + (1.0 - beta0) * sem1
    o1_ref[...] = lax.dot_general(h_ref[...], w1a_ref[...],
                                  (((1,), (1,)), ((), ())),
                                  preferred_element_type=jnp.float32)
    o2_ref[...] = lax.dot_general(ui, w1b_ref[...], (((1,), (1,)), ((), ())),
                                  preferred_element_type=jnp.float32)
    o3_ref[...] = lax.dot_general(ui, w1c_ref[...], (((1,), (1,)), ((), ())),
                                  preferred_element_type=jnp.float32)


def _tc_sem(g0a, g0b, g1a, g1b, b0, b1, saw1, sab1, saw2, h, w1a, w1b, w1c):
    hh = HEADS * HDIM
    return pl.pallas_call(
        _tc_sem_body,
        grid=(GRID,),
        in_specs=[_row_spec(hh), _row_spec(hh), _row_spec(hh), _row_spec(hh),
                  _full_spec((1, hh)), _full_spec((1, hh)),
                  _full_spec((SA_H, hh)), _full_spec((1, SA_H)),
                  _full_spec((1, SA_H)), _row_spec(D),
                  _full_spec((FC_H, D)), _full_spec((FC_H, hh)),
                  _full_spec((FC_H, hh))],
        out_specs=[_row_spec(FC_H), _row_spec(FC_H), _row_spec(FC_H)],
        out_shape=[jax.ShapeDtypeStruct((N, FC_H), jnp.float32)] * 3,
    )(g0a, g0b, g1a, g1b, b0, b1, saw1, sab1, saw2, h, w1a, w1b, w1c)


def _tc_fc_body(sa_ref, sb_ref, sc_ref, b1_ref, w2_ref, b2_ref, o_ref):
    x = sa_ref[...] + sb_ref[...] + sc_ref[...] + b1_ref[...]
    o_ref[...] = (jnp.sum(x * w2_ref[...], axis=1, keepdims=True)
                  + b2_ref[...])


def _tc_fc(sa, sb, sc, b1, w2, b2):
    rb = 512
    return pl.pallas_call(
        _tc_fc_body,
        grid=(BPAIR // rb,),
        in_specs=[pl.BlockSpec((rb, FC_H), lambda i: (i, 0))] * 3
        + [_full_spec((1, FC_H)), _full_spec((1, FC_H)),
           _full_spec((1, 1))],
        out_specs=pl.BlockSpec((rb, 1), lambda i: (i, 0)),
        out_shape=jax.ShapeDtypeStruct((BPAIR, 1), jnp.float32),
    )(sa, sb, sc, b1, w2, b2)


def kernel(feat1, feat2, edge_index_homo, edge_index_mp0, edge_index_mp1,
           pairs, rgcn_gcn_W, rgcn_gcn_b, rgcn_gru_Wih, rgcn_gru_bih,
           rgcn_gru_bhh, han_gat_W, han_attn_l, han_attn_r, han_gat_b,
           sa_W1, sa_b1, sa_W2, fc_W1, fc_b1, fc_W2, fc_b2):
    src_h = edge_index_homo[0]
    dst_h = edge_index_homo[1]

    degs = _sc_degrees(src_h, dst_h)
    do_a, do_b = degs[0, 0], degs[1, 0]
    di_a, di_b = degs[0, 1], degs[1, 1]

    h = feat1
    for i in range(NBLK):
        hp = _tc_gcn_pre(h, do_a, do_b, rgcn_gcn_W[i])
        aggs = _sc_gcn_agg(hp, src_h, dst_h)
        h = _tc_gcn_post(aggs, di_a, di_b, rgcn_gcn_b[i].reshape(1, D),
                         rgcn_gru_Wih[i], rgcn_gru_bih[i].reshape(1, 3 * D),
                         rgcn_gru_bhh[i].reshape(1, 3 * D))
    user_feat = h

    gat_parts = []
    for i, ei in enumerate((edge_index_mp0, edge_index_mp1)):
        src, dst = ei[0], ei[1]
        al = han_attn_l[i]
        ar = han_attn_r[i]
        eye = jnp.eye(HEADS, 16, dtype=jnp.float32)
        al16 = (al[:, :, None] * eye[:, None, :]).reshape(HEADS * HDIM, 16)
        ar16 = (ar[:, :, None] * eye[:, None, :]).reshape(HEADS * HDIM, 16)
        feat, el, er, ml, mr = _tc_gat_proj(feat2, han_gat_W[i], al16, ar16)
        a_e, den_parts = _sc_gat_edge(el, er, ml, mr, src, dst)
        den = _tc_add16(den_parts[0], den_parts[1])
        parts = []
        for s in range(4):
            fs = feat[:, s * D:(s + 1) * D]
            parts.append(_SC_GAT_AGG[s](fs, a_e, den, src, dst))
        ga = jnp.concatenate([p[0] for p in parts], axis=1)
        gb = jnp.concatenate([p[1] for p in parts], axis=1)
        gat_parts.append((ga, gb))

    hh = HEADS * HDIM
    w1a = fc_W1[:, :D]
    w1b = fc_W1[:, D:D + hh]
    w1c = fc_W1[:, D + hh:]
    g1, g2, g3 = _tc_sem(
        gat_parts[0][0], gat_parts[0][1], gat_parts[1][0], gat_parts[1][1],
        han_gat_b[0].reshape(1, hh), han_gat_b[1].reshape(1, hh),
        sa_W1, sa_b1.reshape(1, SA_H), sa_W2.reshape(1, SA_H),
        user_feat, w1a, w1b, w1c)

    p0 = pairs[:, 0]
    p1 = pairs[:, 1]
    g = _sc_pair_gather(g1, g2, g3, p0, p1)
    out = _tc_fc(g[0], g[1], g[2], fc_b1.reshape(1, FC_H),
                 fc_W2.reshape(1, FC_H), fc_b2.reshape(1, 1))
    return out


# speedup vs baseline: 13.1277x; 13.1277x over previous
import functools

import jax
import jax.numpy as jnp
from jax import lax
from jax.experimental import pallas as pl
from jax.experimental.pallas import tpu as pltpu
from jax.experimental.pallas import tpu_sc as plsc

N = 10000
E = 320000
D = 128
HEADS = 8
HDIM = 64
NBLK = 4
BPAIR = 4096
SA_H = 64
FC_H = 128

NC = 2
NS = 16
NW = NC * NS
CH = 80
CHUNKS = E // (NW * CH)
STRIPE = 624
CPH = 208
TAIL0 = NS * STRIPE
TAILN = N - TAIL0

_MESH = plsc.VectorSubcoreMesh(core_axis_name="c", subcore_axis_name="s")


def _worker_id():
    return lax.axis_index("s") * NC + lax.axis_index("c")


def _zero_buf(buf, h):
    def zrow(i, _):
        for j in range(buf.shape[1] // 16):
            buf[i, pl.ds(j * 16, 16)] = jnp.zeros((16,), jnp.float32)
        return 0

    lax.fori_loop(0, h, zrow, 0)


def _zero_fill(zbuf, acc_list):
    _zero_buf(zbuf, CH)
    sid = lax.axis_index("s")
    for acc in acc_list:
        base = sid * STRIPE
        for t in range(7):
            pltpu.sync_copy(zbuf, acc.at[pl.ds(base + t * CH, CH)])
        pltpu.sync_copy(zbuf.at[pl.ds(0, 64)], acc.at[pl.ds(base + 560, 64)])

        @pl.when(sid == NS - 1)
        def _():
            pltpu.sync_copy(zbuf.at[pl.ds(0, TAILN)],
                            acc.at[pl.ds(TAIL0, TAILN)])

    plsc.subcore_barrier()


def _dump_acc(acc, out_ref, dump, cid):
    sid = lax.axis_index("s")
    base = sid * STRIPE
    for t in range(7):
        r0 = base + t * CH
        pltpu.sync_copy(acc.at[pl.ds(r0, CH)], dump)
        pltpu.sync_copy(dump, out_ref.at[cid, pl.ds(r0, CH)])
    pltpu.sync_copy(acc.at[pl.ds(base + 560, 64)], dump.at[pl.ds(0, 64)])
    pltpu.sync_copy(dump.at[pl.ds(0, 64)],
                    out_ref.at[cid, pl.ds(base + 560, 64)])

    @pl.when(sid == NS - 1)
    def _():
        pltpu.sync_copy(acc.at[pl.ds(TAIL0, TAILN)], dump.at[pl.ds(0, TAILN)])
        pltpu.sync_copy(dump.at[pl.ds(0, TAILN)],
                        out_ref.at[cid, pl.ds(TAIL0, TAILN)])


@functools.partial(
    pl.kernel,
    out_type=jax.ShapeDtypeStruct((2, N, D), jnp.float32),
    mesh=_MESH,
    scratch_types=[
        pltpu.VMEM((CH,), jnp.int32),
        pltpu.VMEM((CH,), jnp.int32),
        pltpu.VMEM((CH, D), jnp.float32),
        pltpu.VMEM((CH, D), jnp.float32),
        pltpu.VMEM_SHARED((N, D), jnp.float32),
    ],
)
def _sc_degrees(src_hbm, dst_hbm, out_hbm, src_v, dst_v, ones_l, ones_r,
                acc):
    cid = lax.axis_index("c")
    wid = _worker_id()
    _zero_fill(ones_l, [acc])

    def fill(i, _):
        for j in range(4):
            ones_l[i, pl.ds(j * 16, 16)] = jnp.ones((16,), jnp.float32)
            ones_r[i, pl.ds(j * 16, 16)] = jnp.zeros((16,), jnp.float32)
        for j in range(4, 8):
            ones_l[i, pl.ds(j * 16, 16)] = jnp.zeros((16,), jnp.float32)
            ones_r[i, pl.ds(j * 16, 16)] = jnp.ones((16,), jnp.float32)
        return 0

    lax.fori_loop(0, CH, fill, 0)

    def body(j, _):
        off = (j * NW + wid) * CH
        pltpu.sync_copy(src_hbm.at[pl.ds(off, CH)], src_v)
        pltpu.sync_copy(dst_hbm.at[pl.ds(off, CH)], dst_v)
        pltpu.sync_copy(ones_l, acc.at[src_v], add=True)
        pltpu.sync_copy(ones_r, acc.at[dst_v], add=True)
        return 0

    lax.fori_loop(0, CHUNKS, body, 0)
    plsc.subcore_barrier()
    _dump_acc(acc, out_hbm, ones_l, cid)


@functools.partial(
    pl.kernel,
    out_type=jax.ShapeDtypeStruct((2, N, D), jnp.float32),
    mesh=_MESH,
    scratch_types=[
        pltpu.VMEM((CH,), jnp.int32),
        pltpu.VMEM((CH,), jnp.int32),
        pltpu.VMEM((CH, D), jnp.float32),
        pltpu.VMEM_SHARED((N, D), jnp.float32),
        pltpu.SemaphoreType.DMA,
    ],
)
def _sc_gcn_agg(h_hbm, src_hbm, dst_hbm, out_hbm, src_v, dst_v, rows_v,
                acc, sem):
    cid = lax.axis_index("c")
    wid = _worker_id()
    _zero_fill(rows_v, [acc])

    def body(j, _):
        off = (j * NW + wid) * CH
        pltpu.sync_copy(src_hbm.at[pl.ds(off, CH)], src_v)
        pltpu.sync_copy(dst_hbm.at[pl.ds(off, CH)], dst_v)
        pltpu.async_copy(h_hbm.at[src_v], rows_v, sem).wait()
        pltpu.sync_copy(rows_v, acc.at[dst_v], add=True)
        return 0

    lax.fori_loop(0, CHUNKS, body, 0)
    plsc.subcore_barrier()
    _dump_acc(acc, out_hbm, rows_v, cid)


@functools.partial(
    pl.kernel,
    out_type=(
        jax.ShapeDtypeStruct((E, D), jnp.float32),
        jax.ShapeDtypeStruct((2, N, D), jnp.float32),
    ),
    mesh=_MESH,
    scratch_types=[
        pltpu.VMEM((CH,), jnp.int32),
        pltpu.VMEM((CH,), jnp.int32),
        pltpu.VMEM((CH, D), jnp.float32),
        pltpu.VMEM((CH, D), jnp.float32),
        pltpu.VMEM((CH, D), jnp.float32),
        pltpu.VMEM((8, D), jnp.float32),
        pltpu.VMEM_SHARED((N, D), jnp.float32),
        pltpu.SemaphoreType.DMA,
        pltpu.SemaphoreType.DMA,
    ],
)
def _sc_gat_edge(el_hbm, er_hbm, ml_hbm, mr_hbm, src_hbm, dst_hbm,
                 a_hbm, den_hbm, src_v, dst_v, l_v, r_v, a_v, m_v,
                 acc, sem_l, sem_r):
    cid = lax.axis_index("c")
    wid = _worker_id()
    _zero_fill(a_v, [acc])
    pltpu.sync_copy(ml_hbm, m_v.at[pl.ds(0, 1)])
    pltpu.sync_copy(mr_hbm, m_v.at[pl.ds(1, 1)])
    msum = m_v[0, pl.ds(0, 16)] + m_v[1, pl.ds(0, 16)]
    cvec = jnp.maximum(msum, 0.2 * msum)

    def body(j, _):
        off = (j * NW + wid) * CH
        pltpu.sync_copy(src_hbm.at[pl.ds(off, CH)], src_v)
        pltpu.sync_copy(dst_hbm.at[pl.ds(off, CH)], dst_v)
        cl = pltpu.async_copy(el_hbm.at[src_v], l_v, sem_l)
        cr = pltpu.async_copy(er_hbm.at[dst_v], r_v, sem_r)
        cl.wait()
        cr.wait()

        def edge(k, _):
            v = l_v[k, pl.ds(0, 16)] + r_v[k, pl.ds(0, 16)]
            e = jnp.maximum(v, 0.2 * v)
            a_v[k, pl.ds(0, 16)] = jnp.exp(e - cvec)
            return 0

        lax.fori_loop(0, CH, edge, 0)
        pltpu.sync_copy(a_v, acc.at[dst_v], add=True)
        pltpu.sync_copy(a_v, a_hbm.at[pl.ds(off, CH)])
        return 0

    lax.fori_loop(0, CHUNKS, body, 0)
    plsc.subcore_barrier()
    _dump_acc(acc, den_hbm, l_v, cid)


def _make_sc_gat_agg(h0):
    @functools.partial(
        pl.kernel,
        out_type=jax.ShapeDtypeStruct((2, N, D), jnp.float32),
        mesh=_MESH,
        scratch_types=[
            pltpu.VMEM((CH,), jnp.int32),
            pltpu.VMEM((CH,), jnp.int32),
            pltpu.VMEM((CH, D), jnp.float32),
            pltpu.VMEM((CH, D), jnp.float32),
            pltpu.VMEM_SHARED((N, D), jnp.float32),
            pltpu.SemaphoreType.DMA,
        ],
    )
    def _sc_gat_agg(feat_hbm, a_hbm, src_hbm, dst_hbm, out_hbm,
                    src_v, dst_v, rows_v, a_v, acc, sem):
        cid = lax.axis_index("c")
        wid = _worker_id()
        _zero_fill(rows_v, [acc])

        def body(j, _):
            off = (j * NW + wid) * CH
            pltpu.sync_copy(src_hbm.at[pl.ds(off, CH)], src_v)
            pltpu.sync_copy(dst_hbm.at[pl.ds(off, CH)], dst_v)
            pltpu.sync_copy(a_hbm.at[pl.ds(off, CH)], a_v)
            pltpu.async_copy(feat_hbm.at[src_v], rows_v, sem).wait()

            def edge(k, _):
                av = a_v[k, pl.ds(0, 16)]
                al0 = av[h0]
                al1 = av[h0 + 1]
                for q in range(4):
                    sl = pl.ds(q * 16, 16)
                    rows_v[k, sl] = rows_v[k, sl] * al0
                for q in range(4, 8):
                    sl = pl.ds(q * 16, 16)
                    rows_v[k, sl] = rows_v[k, sl] * al1
                return 0

            lax.fori_loop(0, CH, edge, 0)
            pltpu.sync_copy(rows_v, acc.at[dst_v], add=True)
            return 0

        lax.fori_loop(0, CHUNKS, body, 0)
        plsc.subcore_barrier()
        _dump_acc(acc, out_hbm, rows_v, cid)

    return _sc_gat_agg


_SC_GAT_AGG = [_make_sc_gat_agg(2 * s) for s in range(4)]


PB = BPAIR // NW


@functools.partial(
    pl.kernel,
    out_type=jax.ShapeDtypeStruct((3, BPAIR, D), jnp.float32),
    mesh=_MESH,
    scratch_types=[
        pltpu.VMEM((PB,), jnp.int32),
        pltpu.VMEM((PB,), jnp.int32),
        pltpu.VMEM((PB, D), jnp.float32),
        pltpu.SemaphoreType.DMA,
    ],
)
def _sc_pair_gather(g1_hbm, g2_hbm, g3_hbm, p0_hbm, p1_hbm, out_hbm,
                    i0_v, i1_v, rows_v, sem):
    wid = _worker_id()
    base = wid * PB
    pltpu.sync_copy(p0_hbm.at[pl.ds(base, PB)], i0_v)
    pltpu.sync_copy(p1_hbm.at[pl.ds(base, PB)], i1_v)
    pltpu.async_copy(g1_hbm.at[i0_v], rows_v, sem).wait()
    pltpu.sync_copy(rows_v, out_hbm.at[0, pl.ds(base, PB)])
    pltpu.async_copy(g2_hbm.at[i0_v], rows_v, sem).wait()
    pltpu.sync_copy(rows_v, out_hbm.at[1, pl.ds(base, PB)])
    pltpu.async_copy(g3_hbm.at[i1_v], rows_v, sem).wait()
    pltpu.sync_copy(rows_v, out_hbm.at[2, pl.ds(base, PB)])


RB = 1000
GRID = N // RB


def _row_spec(w):
    return pl.BlockSpec((RB, w), lambda i: (i, 0))


def _full_spec(shape):
    return pl.BlockSpec(shape, lambda i: tuple(0 for _ in shape))


def _norm(deg):
    return jnp.where(deg > 0, lax.rsqrt(jnp.maximum(deg, 1e-9)), 0.0)


def _tc_gcn_pre_body(h_ref, da_ref, db_ref, w_ref, o_ref):
    ns = _norm(da_ref[:, 0:1] + db_ref[:, 0:1])
    o_ref[...] = lax.dot_general(h_ref[...] * ns, w_ref[...],
                                 (((1,), (1,)), ((), ())),
                                 preferred_element_type=jnp.float32)


def _tc_gcn_pre(h, dega, degb, w):
    return pl.pallas_call(
        _tc_gcn_pre_body,
        grid=(GRID,),
        in_specs=[_row_spec(D), _row_spec(D), _row_spec(D),
                  _full_spec((D, D))],
        out_specs=_row_spec(D),
        out_shape=jax.ShapeDtypeStruct((N, D), jnp.float32),
    )(h, dega, degb, w)


def _tc_gcn_post_body(aa_ref, ab_ref, da_ref, db_ref, b_ref, wih_ref,
                      bih_ref, bhh_ref, o_ref):
    nd = _norm(da_ref[:, 64:65] + db_ref[:, 64:65])
    x = (aa_ref[...] + ab_ref[...]) * nd + b_ref[...]
    gi = lax.dot_general(x, wih_ref[...], (((1,), (1,)), ((), ())),
                         preferred_element_type=jnp.float32) + bih_ref[...]
    bhh = bhh_ref[...]
    r = jax.nn.sigmoid(gi[:, 0:D] + bhh[:, 0:D])
    z = jax.nn.sigmoid(gi[:, D:2 * D] + bhh[:, D:2 * D])
    n = jnp.tanh(gi[:, 2 * D:3 * D] + r * bhh[:, 2 * D:3 * D])
    o_ref[...] = jnp.maximum((1.0 - z) * n, 0.0)


def _tc_gcn_post(aggs, dega, degb, b, wih, bih, bhh):
    return pl.pallas_call(
        _tc_gcn_post_body,
        grid=(GRID,),
        in_specs=[_row_spec(D), _row_spec(D), _row_spec(D), _row_spec(D),
                  _full_spec((1, D)), _full_spec((3 * D, D)),
                  _full_spec((1, 3 * D)), _full_spec((1, 3 * D))],
        out_specs=_row_spec(D),
        out_shape=jax.ShapeDtypeStruct((N, D), jnp.float32),
    )(aggs[0], aggs[1], dega, degb, b, wih, bih, bhh)


def _tc_gat_proj_body(f_ref, w_ref, al_ref, ar_ref, feat_ref, el_ref,
                      er_ref, ml_ref, mr_ref):
    feat = lax.dot_general(f_ref[...], w_ref[...], (((1,), (1,)), ((), ())),
                           preferred_element_type=jnp.float32)
    feat_ref[...] = feat
    el = jnp.dot(feat, al_ref[...], preferred_element_type=jnp.float32)
    er = jnp.dot(feat, ar_ref[...], preferred_element_type=jnp.float32)
    el_ref[...] = el
    er_ref[...] = er
    ml = jnp.max(el, axis=0, keepdims=True)
    mr = jnp.max(er, axis=0, keepdims=True)

    @pl.when(pl.program_id(0) == 0)
    def _():
        ml_ref[...] = ml
        mr_ref[...] = mr

    @pl.when(pl.program_id(0) > 0)
    def _():
        ml_ref[...] = jnp.maximum(ml_ref[...], ml)
        mr_ref[...] = jnp.maximum(mr_ref[...], mr)


def _tc_gat_proj(f2, w, al128, ar128):
    return pl.pallas_call(
        _tc_gat_proj_body,
        grid=(GRID,),
        in_specs=[_row_spec(D), _full_spec((HEADS * HDIM, D)),
                  _full_spec((HEADS * HDIM, D)),
                  _full_spec((HEADS * HDIM, D))],
        out_specs=[_row_spec(HEADS * HDIM), _row_spec(D), _row_spec(D),
                   _full_spec((1, D)), _full_spec((1, D))],
        out_shape=[
            jax.ShapeDtypeStruct((N, HEADS * HDIM), jnp.float32),
            jax.ShapeDtypeStruct((N, D), jnp.float32),
            jax.ShapeDtypeStruct((N, D), jnp.float32),
            jax.ShapeDtypeStruct((1, D), jnp.float32),
            jax.ShapeDtypeStruct((1, D), jnp.float32),
        ],
    )(f2, w, al128, ar128)


def _tc_sem_body(u0a_ref, u0b_ref, u1a_ref, u1b_ref, d0a_ref, d0b_ref,
                 d1a_ref, d1b_ref, rep_ref, b0_ref, b1_ref,
                 w1_ref, sb1_ref, w2_ref, h_ref, w1a_ref, w1b_ref, w1c_ref,
                 o1_ref, o2_ref, o3_ref):
    def elu(x):
        return jnp.where(x > 0, x, jnp.exp(jnp.minimum(x, 0.0)) - 1.0)

    def semantic(ua, ub, da, db, b):
        den = jnp.dot(da + db, rep_ref[...],
                      preferred_element_type=jnp.float32)
        den = jnp.where(den > 0, den, 1.0)
        return elu((ua + ub) / den + b)

    sem0 = semantic(u0a_ref[...], u0b_ref[...], d0a_ref[...], d0b_ref[...],
                    b0_ref[...])
    sem1 = semantic(u1a_ref[...], u1b_ref[...], d1a_ref[...], d1b_ref[...],
                    b1_ref[...])

    def score(s):
        t = jnp.tanh(lax.dot_general(s, w1_ref[...], (((1,), (1,)), ((), ())),
                                     preferred_element_type=jnp.float32)
                     + sb1_ref[...])
        return jnp.sum(t * w2_ref[...], axis=1, keepdims=True)

    w0 = score(sem0)
    w1 = score(sem1)
    m = jnp.maximum(w0, w1)
    e0 = jnp.exp(w0 - m)
    e1 = jnp.exp(w1 - m)
    beta0 = e0 / (e0 + e1)
    ui = beta0 * sem0 + (1.0 - beta0) * sem1
    o1_ref[...] = lax.dot_general(h_ref[...], w1a_ref[...],
                                  (((1,), (1,)), ((), ())),
                                  preferred_element_type=jnp.float32)
    o2_ref[...] = lax.dot_general(ui, w1b_ref[...], (((1,), (1,)), ((), ())),
                                  preferred_element_type=jnp.float32)
    o3_ref[...] = lax.dot_general(ui, w1c_ref[...], (((1,), (1,)), ((), ())),
                                  preferred_element_type=jnp.float32)


def _tc_sem(u0a, u0b, u1a, u1b, d0a, d0b, d1a, d1b, rep, b0, b1, saw1, sab1,
            saw2, h, w1a, w1b, w1c):
    hh = HEADS * HDIM
    return pl.pallas_call(
        _tc_sem_body,
        grid=(GRID,),
        in_specs=[_row_spec(hh), _row_spec(hh), _row_spec(hh), _row_spec(hh),
                  _row_spec(D), _row_spec(D), _row_spec(D), _row_spec(D),
                  _full_spec((D, hh)),
                  _full_spec((1, hh)), _full_spec((1, hh)),
                  _full_spec((SA_H, hh)), _full_spec((1, SA_H)),
                  _full_spec((1, SA_H)), _row_spec(D),
                  _full_spec((FC_H, D)), _full_spec((FC_H, hh)),
                  _full_spec((FC_H, hh))],
        out_specs=[_row_spec(FC_H), _row_spec(FC_H), _row_spec(FC_H)],
        out_shape=[jax.ShapeDtypeStruct((N, FC_H), jnp.float32)] * 3,
    )(u0a, u0b, u1a, u1b, d0a, d0b, d1a, d1b, rep, b0, b1, saw1, sab1,
      saw2, h, w1a, w1b, w1c)


def _tc_fc_body(sa_ref, sb_ref, sc_ref, b1_ref, w2_ref, b2_ref, o_ref):
    x = sa_ref[...] + sb_ref[...] + sc_ref[...] + b1_ref[...]
    o_ref[...] = (jnp.sum(x * w2_ref[...], axis=1, keepdims=True)
                  + b2_ref[...])


def _tc_fc(sa, sb, sc, b1, w2, b2):
    rb = 512
    return pl.pallas_call(
        _tc_fc_body,
        grid=(BPAIR // rb,),
        in_specs=[pl.BlockSpec((rb, FC_H), lambda i: (i, 0))] * 3
        + [_full_spec((1, FC_H)), _full_spec((1, FC_H)),
           _full_spec((1, 1))],
        out_specs=pl.BlockSpec((rb, 1), lambda i: (i, 0)),
        out_shape=jax.ShapeDtypeStruct((BPAIR, 1), jnp.float32),
    )(sa, sb, sc, b1, w2, b2)


def kernel(feat1, feat2, edge_index_homo, edge_index_mp0, edge_index_mp1,
           pairs, rgcn_gcn_W, rgcn_gcn_b, rgcn_gru_Wih, rgcn_gru_bih,
           rgcn_gru_bhh, han_gat_W, han_attn_l, han_attn_r, han_gat_b,
           sa_W1, sa_b1, sa_W2, fc_W1, fc_b1, fc_W2, fc_b2):
    src_h = edge_index_homo[0]
    dst_h = edge_index_homo[1]

    degs = _sc_degrees(src_h, dst_h)
    deg_a, deg_b = degs[0], degs[1]

    h = feat1
    for i in range(NBLK):
        hp = _tc_gcn_pre(h, deg_a, deg_b, rgcn_gcn_W[i])
        aggs = _sc_gcn_agg(hp, src_h, dst_h)
        h = _tc_gcn_post(aggs, deg_a, deg_b, rgcn_gcn_b[i].reshape(1, D),
                         rgcn_gru_Wih[i], rgcn_gru_bih[i].reshape(1, 3 * D),
                         rgcn_gru_bhh[i].reshape(1, 3 * D))
    user_feat = h

    hh = HEADS * HDIM
    eye = jnp.eye(HEADS, D, dtype=jnp.float32)
    gat_u = []
    gat_d = []
    for i, ei in enumerate((edge_index_mp0, edge_index_mp1)):
        src, dst = ei[0], ei[1]
        al = han_attn_l[i]
        ar = han_attn_r[i]
        al128 = (al[:, :, None] * eye[:, None, :]).reshape(hh, D)
        ar128 = (ar[:, :, None] * eye[:, None, :]).reshape(hh, D)
        feat, el, er, ml, mr = _tc_gat_proj(feat2, han_gat_W[i], al128, ar128)
        a_e, den_parts = _sc_gat_edge(el, er, ml, mr, src, dst)
        parts = []
        for s in range(4):
            fs = feat[:, s * D:(s + 1) * D]
            parts.append(_SC_GAT_AGG[s](fs, a_e, src, dst))
        gat_u.append((jnp.concatenate([p[0] for p in parts], axis=1),
                      jnp.concatenate([p[1] for p in parts], axis=1)))
        gat_d.append(den_parts)

    rep = (eye.T[:, :, None] * jnp.ones((1, 1, HDIM))).reshape(D, hh)
    w1a = fc_W1[:, :D]
    w1b = fc_W1[:, D:D + hh]
    w1c = fc_W1[:, D + hh:]
    g1, g2, g3 = _tc_sem(
        gat_u[0][0], gat_u[0][1], gat_u[1][0], gat_u[1][1],
        gat_d[0][0], gat_d[0][1], gat_d[1][0], gat_d[1][1], rep,
        han_gat_b[0].reshape(1, hh), han_gat_b[1].reshape(1, hh),
        sa_W1, sa_b1.reshape(1, SA_H), sa_W2.reshape(1, SA_H),
        user_feat, w1a, w1b, w1c)

    p0 = pairs[:, 0]
    p1 = pairs[:, 1]
    g = _sc_pair_gather(g1, g2, g3, p0, p1)
    out = _tc_fc(g[0], g[1], g[2], fc_b1.reshape(1, FC_H),
                 fc_W2.reshape(1, FC_H), fc_b2.reshape(1, 1))
    return out


# speedup vs baseline: 24.7768x; 1.8874x over previous
import functools

import jax
import jax.numpy as jnp
from jax import lax
from jax.experimental import pallas as pl
from jax.experimental.pallas import tpu as pltpu
from jax.experimental.pallas import tpu_sc as plsc

N = 10000
E = 320000
D = 128
HEADS = 8
HDIM = 64
NBLK = 4
BPAIR = 4096
SA_H = 64
FC_H = 128

NC = 2
NS = 16
NW = NC * NS
CH = 80
CHUNKS = E // (NW * CH)
NBUF = 5
PK = CH // 8
PR = 16
STRIPE = 624
TAIL0 = NS * STRIPE
TAILN = N - TAIL0

_MESH = plsc.VectorSubcoreMesh(core_axis_name="c", subcore_axis_name="s")


def _worker_id():
    return lax.axis_index("s") * NC + lax.axis_index("c")


def _zero_buf(buf, h):
    def zrow(i, _):
        for j in range(buf.shape[-1] // 16):
            buf[i, pl.ds(j * 16, 16)] = jnp.zeros((16,), jnp.float32)
        return 0

    lax.fori_loop(0, h, zrow, 0)


def _zero_fill(zbuf, acc_list):
    _zero_buf(zbuf, CH)
    sid = lax.axis_index("s")
    for acc in acc_list:
        base = sid * STRIPE
        for t in range(7):
            pltpu.sync_copy(zbuf, acc.at[pl.ds(base + t * CH, CH)])
        pltpu.sync_copy(zbuf.at[pl.ds(0, 64)], acc.at[pl.ds(base + 560, 64)])

        @pl.when(sid == NS - 1)
        def _():
            pltpu.sync_copy(zbuf.at[pl.ds(0, TAILN)],
                            acc.at[pl.ds(TAIL0, TAILN)])

    plsc.subcore_barrier()


def _dump_acc(acc, out_ref, dump, cid):
    sid = lax.axis_index("s")
    base = sid * STRIPE
    for t in range(7):
        r0 = base + t * CH
        pltpu.sync_copy(acc.at[pl.ds(r0, CH)], dump)
        pltpu.sync_copy(dump, out_ref.at[cid, pl.ds(r0, CH)])
    pltpu.sync_copy(acc.at[pl.ds(base + 560, 64)], dump.at[pl.ds(0, 64)])
    pltpu.sync_copy(dump.at[pl.ds(0, 64)],
                    out_ref.at[cid, pl.ds(base + 560, 64)])

    @pl.when(sid == NS - 1)
    def _():
        pltpu.sync_copy(acc.at[pl.ds(TAIL0, TAILN)], dump.at[pl.ds(0, TAILN)])
        pltpu.sync_copy(dump.at[pl.ds(0, TAILN)],
                        out_ref.at[cid, pl.ds(TAIL0, TAILN)])


SPLITS3 = ((0, 40, 2), (40, 40, 2), (80, 45, 3))
SPLITS_F = ((0, 63, 3), (63, 62, 2))
IDXR = 45


@functools.partial(
    pl.kernel,
    out_type=jax.ShapeDtypeStruct((2, N, D), jnp.float32),
    mesh=_MESH,
    scratch_types=[
        pltpu.VMEM((IDXR, CH), jnp.int32),
        pltpu.VMEM((IDXR, CH), jnp.int32),
        pltpu.VMEM((CH, D), jnp.float32),
        pltpu.VMEM((CH, D), jnp.float32),
        pltpu.VMEM_SHARED((N, D), jnp.float32),
        pltpu.SemaphoreType.DMA,
    ],
)
def _sc_degrees(src_hbm, dst_hbm, out_hbm, si_v, di_v, ones_l, ones_r,
                acc, sem_s):
    cid = lax.axis_index("c")
    wid = _worker_id()
    _zero_fill(ones_l, [acc])

    def fill(i, _):
        for j in range(4):
            ones_l[i, pl.ds(j * 16, 16)] = jnp.ones((16,), jnp.float32)
            ones_r[i, pl.ds(j * 16, 16)] = jnp.zeros((16,), jnp.float32)
        for j in range(4, 8):
            ones_l[i, pl.ds(j * 16, 16)] = jnp.zeros((16,), jnp.float32)
            ones_r[i, pl.ds(j * 16, 16)] = jnp.ones((16,), jnp.float32)
        return 0

    lax.fori_loop(0, CH, fill, 0)

    for base, cnt, _unused in SPLITS3:
        pltpu.sync_copy(src_hbm.at[wid, pl.ds(base, cnt)],
                        si_v.at[pl.ds(0, cnt)])
        pltpu.sync_copy(dst_hbm.at[wid, pl.ds(base, cnt)],
                        di_v.at[pl.ds(0, cnt)])

        @pl.loop(0, cnt, step=5)
        def _(g):
            for b in range(5):
                j = g + b
                pltpu.async_copy(ones_l, acc.at[si_v.at[j]], sem_s,
                                 add=True)
                pltpu.async_copy(ones_r, acc.at[di_v.at[j]], sem_s,
                                 add=True)

        @pl.loop(0, 2 * cnt, step=5)
        def _(g):
            for b in range(5):
                pltpu.make_async_copy(ones_l, acc.at[si_v.at[0]],
                                      sem_s).wait()

    plsc.subcore_barrier()
    _dump_acc(acc, out_hbm, ones_l, cid)


@functools.partial(
    pl.kernel,
    out_type=jax.ShapeDtypeStruct((2, N, D), jnp.float32),
    mesh=_MESH,
    scratch_types=[
        pltpu.VMEM((IDXR, CH), jnp.int32),
        pltpu.VMEM((IDXR, CH), jnp.int32),
        pltpu.VMEM((3, CH, D), jnp.float32),
        pltpu.VMEM_SHARED((N, D), jnp.float32),
        pltpu.SemaphoreType.DMA,
        pltpu.SemaphoreType.DMA,
    ],
)
def _sc_gcn_agg(h_hbm, src_hbm, dst_hbm, out_hbm, si_v, di_v, rows_v,
                acc, sem_g, sem_s):
    cid = lax.axis_index("c")
    wid = _worker_id()
    _zero_fill(rows_v.at[0], [acc])

    def wait_g():
        pltpu.make_async_copy(h_hbm.at[si_v.at[0]], rows_v.at[0],
                              sem_g).wait()

    def wait_s():
        pltpu.make_async_copy(rows_v.at[0], acc.at[di_v.at[0]],
                              sem_s).wait()

    for base, cnt, R in SPLITS3:
        pltpu.sync_copy(src_hbm.at[wid, pl.ds(base, cnt)],
                        si_v.at[pl.ds(0, cnt)])
        pltpu.sync_copy(dst_hbm.at[wid, pl.ds(base, cnt)],
                        di_v.at[pl.ds(0, cnt)])
        for p in range(R - 1):
            pltpu.async_copy(h_hbm.at[si_v.at[p]], rows_v.at[p], sem_g)

        @pl.loop(0, cnt, step=R)
        def _(g):
            for b in range(R):
                j = g + b
                wait_g()
                pltpu.async_copy(rows_v.at[b], acc.at[di_v.at[j]], sem_s,
                                 add=True)

                @pl.when(j + R - 1 < cnt)
                def _():
                    @pl.when(j >= 1)
                    def _():
                        wait_s()

                    pltpu.async_copy(h_hbm.at[si_v.at[j + R - 1]],
                                     rows_v.at[(b - 1) % R], sem_g)

        for _unused in range(R):
            wait_s()

    plsc.subcore_barrier()
    _dump_acc(acc, out_hbm, rows_v.at[0], cid)


@functools.partial(
    pl.kernel,
    out_type=jax.ShapeDtypeStruct((E // CH * PR, D), jnp.float32),
    mesh=_MESH,
    scratch_types=[
        pltpu.VMEM((CHUNKS, CH), jnp.int32),
        pltpu.VMEM((CHUNKS, CH), jnp.int32),
        pltpu.VMEM((3, CH, D), jnp.float32),
        pltpu.VMEM((3, CH, D), jnp.float32),
        pltpu.VMEM((3, PR, D), jnp.float32),
        pltpu.VMEM((8, D), jnp.float32),
        pltpu.SemaphoreType.DMA,
        pltpu.SemaphoreType.DMA,
        pltpu.SemaphoreType.DMA,
    ],
)
def _sc_gat_edge(el_hbm, er_hbm, ml_hbm, mr_hbm, src_hbm, dst_hbm,
                 a_hbm, si_v, di_v, l_v, r_v, ap_v, m_v,
                 sem_l, sem_r, sem_o):
    wid = _worker_id()
    for b in range(3):
        _zero_buf(ap_v.at[b], PR)
    pltpu.sync_copy(src_hbm.at[wid], si_v)
    pltpu.sync_copy(dst_hbm.at[wid], di_v)
    pltpu.sync_copy(ml_hbm, m_v.at[pl.ds(0, 1)])
    pltpu.sync_copy(mr_hbm, m_v.at[pl.ds(1, 1)])
    msum = m_v[0, pl.ds(0, 16)] + m_v[1, pl.ds(0, 16)]
    cvec = jnp.maximum(msum, 0.2 * msum)

    def arow(j):
        return (wid * CHUNKS + j) * PR

    def fire(j, b):
        pltpu.async_copy(el_hbm.at[si_v.at[j]], l_v.at[b], sem_l)
        pltpu.async_copy(er_hbm.at[di_v.at[j]], r_v.at[b], sem_r)

    def wait_lr():
        pltpu.make_async_copy(el_hbm.at[si_v.at[0]], l_v.at[0],
                              sem_l).wait()
        pltpu.make_async_copy(er_hbm.at[di_v.at[0]], r_v.at[0],
                              sem_r).wait()

    def wait_o():
        pltpu.make_async_copy(ap_v.at[0], a_hbm.at[pl.ds(0, PR)],
                              sem_o).wait()

    for base, cnt, R in SPLITS_F:
        for p in range(R - 1):
            fire(base + p, p)

        @pl.loop(0, cnt, step=R)
        def _(g):
            for b in range(R):
                j = g + b
                wait_lr()

                @pl.when(j >= R)
                def _():
                    wait_o()

                def edge_grp(g2, _carry):
                    for u in range(8):
                        k = g2 * 8 + u
                        v = (l_v[b, k, pl.ds(0, 16)]
                             + r_v[b, k, pl.ds(0, 16)])
                        e = jnp.maximum(v, 0.2 * v)
                        ap_v[b, g2, pl.ds(u * 16, 16)] = jnp.exp(e - cvec)
                    return 0

                lax.fori_loop(0, PK, edge_grp, 0)

                @pl.when(j + R - 1 < cnt)
                def _():
                    fire(base + j + R - 1, (b - 1) % R)

                pltpu.async_copy(ap_v.at[b],
                                 a_hbm.at[pl.ds(arow(base + j), PR)],
                                 sem_o)

        for _unused in range(R):
            wait_o()


@functools.partial(
    pl.kernel,
    out_type=jax.ShapeDtypeStruct((2, N, D), jnp.float32),
    mesh=_MESH,
    scratch_types=[
        pltpu.VMEM((IDXR, CH), jnp.int32),
        pltpu.VMEM((3, CH, D), jnp.float32),
        pltpu.VMEM((3, PR, D), jnp.float32),
        pltpu.VMEM_SHARED((N, D), jnp.float32),
        pltpu.SemaphoreType.DMA,
        pltpu.SemaphoreType.DMA,
    ],
)
def _sc_gat_den(a_hbm, dst_hbm, out_hbm, di_v, a_v, ap_v, acc,
                sem_a, sem_s):
    cid = lax.axis_index("c")
    wid = _worker_id()
    _zero_fill(a_v.at[0], [acc])
    _zero_buf(a_v.at[1], CH)
    _zero_buf(a_v.at[2], CH)

    def arow(j):
        return (wid * CHUNKS + j) * PR

    def wait_a():
        pltpu.make_async_copy(a_hbm.at[pl.ds(0, PR)], ap_v.at[0],
                              sem_a).wait()

    def wait_s():
        pltpu.make_async_copy(a_v.at[0], acc.at[di_v.at[0]],
                              sem_s).wait()

    for base, cnt, R in SPLITS3:
        pltpu.sync_copy(dst_hbm.at[wid, pl.ds(base, cnt)],
                        di_v.at[pl.ds(0, cnt)])
        for p in range(R - 1):
            pltpu.async_copy(a_hbm.at[pl.ds(arow(base + p), PR)],
                             ap_v.at[p], sem_a)

        @pl.loop(0, cnt, step=R)
        def _(g):
            for b in range(R):
                j = g + b
                wait_a()

                @pl.when(j >= R)
                def _():
                    wait_s()

                def edge_grp(g2, _carry):
                    for u in range(8):
                        k = g2 * 8 + u
                        a_v[b, k, pl.ds(0, 16)] = ap_v[b, g2,
                                                       pl.ds(u * 16, 16)]
                    return 0

                lax.fori_loop(0, PK, edge_grp, 0)

                @pl.when(j + R - 1 < cnt)
                def _():
                    pltpu.async_copy(
                        a_hbm.at[pl.ds(arow(base + j + R - 1), PR)],
                        ap_v.at[(b - 1) % R], sem_a)

                pltpu.async_copy(a_v.at[b], acc.at[di_v.at[j]], sem_s,
                                 add=True)

        for _unused in range(R):
            wait_s()

    plsc.subcore_barrier()
    _dump_acc(acc, out_hbm, a_v.at[0], cid)


def _make_sc_gat_agg(h0):
    @functools.partial(
        pl.kernel,
        out_type=jax.ShapeDtypeStruct((2, N, D), jnp.float32),
        mesh=_MESH,
        scratch_types=[
            pltpu.VMEM((IDXR, CH), jnp.int32),
            pltpu.VMEM((IDXR, CH), jnp.int32),
            pltpu.VMEM((3, CH, D), jnp.float32),
            pltpu.VMEM((3, PR, D), jnp.float32),
            pltpu.VMEM_SHARED((N, D), jnp.float32),
            pltpu.SemaphoreType.DMA,
            pltpu.SemaphoreType.DMA,
            pltpu.SemaphoreType.DMA,
        ],
    )
    def _sc_gat_agg(feat_hbm, a_hbm, src_hbm, dst_hbm, out_hbm,
                    si_v, di_v, rows_v, ap_v, acc, sem_g, sem_a, sem_s):
        cid = lax.axis_index("c")
        wid = _worker_id()
        _zero_fill(rows_v.at[0], [acc])

        def arow(j):
            return (wid * CHUNKS + j) * PR

        def wait_ga():
            pltpu.make_async_copy(feat_hbm.at[si_v.at[0]], rows_v.at[0],
                                  sem_g).wait()
            pltpu.make_async_copy(a_hbm.at[pl.ds(0, PR)], ap_v.at[0],
                                  sem_a).wait()

        def wait_s():
            pltpu.make_async_copy(rows_v.at[0], acc.at[di_v.at[0]],
                                  sem_s).wait()

        for base, cnt, R in SPLITS3:
            pltpu.sync_copy(src_hbm.at[wid, pl.ds(base, cnt)],
                            si_v.at[pl.ds(0, cnt)])
            pltpu.sync_copy(dst_hbm.at[wid, pl.ds(base, cnt)],
                            di_v.at[pl.ds(0, cnt)])
            for p in range(R - 1):
                pltpu.async_copy(feat_hbm.at[si_v.at[p]], rows_v.at[p],
                                 sem_g)
                pltpu.async_copy(a_hbm.at[pl.ds(arow(base + p), PR)],
                                 ap_v.at[p], sem_a)

            @pl.loop(0, cnt, step=R)
            def _(g):
                for b in range(R):
                    j = g + b
                    wait_ga()

                    def edge_grp(g2, _carry):
                        for u in range(8):
                            k = g2 * 8 + u
                            av = ap_v[b, g2, pl.ds(u * 16, 16)]
                            al0 = av[h0]
                            al1 = av[h0 + 1]
                            for q in range(4):
                                sl = pl.ds(q * 16, 16)
                                rows_v[b, k, sl] = rows_v[b, k, sl] * al0
                            for q in range(4, 8):
                                sl = pl.ds(q * 16, 16)
                                rows_v[b, k, sl] = rows_v[b, k, sl] * al1
                        return 0

                    lax.fori_loop(0, PK, edge_grp, 0)
                    pltpu.async_copy(rows_v.at[b], acc.at[di_v.at[j]],
                                     sem_s, add=True)

                    @pl.when(j + R - 1 < cnt)
                    def _():
                        @pl.when(j >= 1)
                        def _():
                            wait_s()

                        jn = j + R - 1
                        bn = (b - 1) % R
                        pltpu.async_copy(feat_hbm.at[si_v.at[jn]],
                                         rows_v.at[bn], sem_g)
                        pltpu.async_copy(
                            a_hbm.at[pl.ds(arow(base + jn), PR)],
                            ap_v.at[bn], sem_a)

            for _unused in range(R):
                wait_s()

        plsc.subcore_barrier()
        _dump_acc(acc, out_hbm, rows_v.at[0], cid)

    return _sc_gat_agg


_SC_GAT_AGG = [_make_sc_gat_agg(2 * s) for s in range(4)]


PB = BPAIR // NW


@functools.partial(
    pl.kernel,
    out_type=jax.ShapeDtypeStruct((3, BPAIR, D), jnp.float32),
    mesh=_MESH,
    scratch_types=[
        pltpu.VMEM((PB,), jnp.int32),
        pltpu.VMEM((PB,), jnp.int32),
        pltpu.VMEM((PB, D), jnp.float32),
        pltpu.SemaphoreType.DMA,
    ],
)
def _sc_pair_gather(g1_hbm, g2_hbm, g3_hbm, p0_hbm, p1_hbm, out_hbm,
                    i0_v, i1_v, rows_v, sem):
    wid = _worker_id()
    base = wid * PB
    pltpu.sync_copy(p0_hbm.at[pl.ds(base, PB)], i0_v)
    pltpu.sync_copy(p1_hbm.at[pl.ds(base, PB)], i1_v)
    pltpu.async_copy(g1_hbm.at[i0_v], rows_v, sem).wait()
    pltpu.sync_copy(rows_v, out_hbm.at[0, pl.ds(base, PB)])
    pltpu.async_copy(g2_hbm.at[i0_v], rows_v, sem).wait()
    pltpu.sync_copy(rows_v, out_hbm.at[1, pl.ds(base, PB)])
    pltpu.async_copy(g3_hbm.at[i1_v], rows_v, sem).wait()
    pltpu.sync_copy(rows_v, out_hbm.at[2, pl.ds(base, PB)])


RB = 1000
GRID = N // RB


def _row_spec(w):
    return pl.BlockSpec((RB, w), lambda i: (i, 0))


def _full_spec(shape):
    return pl.BlockSpec(shape, lambda i: tuple(0 for _ in shape))


def _norm(deg):
    return jnp.where(deg > 0, lax.rsqrt(jnp.maximum(deg, 1e-9)), 0.0)


def _tc_gcn_pre_body(h_ref, da_ref, db_ref, w_ref, o_ref):
    ns = _norm(da_ref[:, 0:1] + db_ref[:, 0:1])
    o_ref[...] = lax.dot_general(h_ref[...] * ns, w_ref[...],
                                 (((1,), (1,)), ((), ())),
                                 preferred_element_type=jnp.float32)


def _tc_gcn_pre(h, dega, degb, w):
    return pl.pallas_call(
        _tc_gcn_pre_body,
        grid=(GRID,),
        in_specs=[_row_spec(D), _row_spec(D), _row_spec(D),
                  _full_spec((D, D))],
        out_specs=_row_spec(D),
        out_shape=jax.ShapeDtypeStruct((N, D), jnp.float32),
    )(h, dega, degb, w)


def _tc_gcn_post_body(aa_ref, ab_ref, da_ref, db_ref, b_ref, wih_ref,
                      bih_ref, bhh_ref, o_ref):
    nd = _norm(da_ref[:, 64:65] + db_ref[:, 64:65])
    x = (aa_ref[...] + ab_ref[...]) * nd + b_ref[...]
    gi = lax.dot_general(x, wih_ref[...], (((1,), (1,)), ((), ())),
                         preferred_element_type=jnp.float32) + bih_ref[...]
    bhh = bhh_ref[...]
    r = jax.nn.sigmoid(gi[:, 0:D] + bhh[:, 0:D])
    z = jax.nn.sigmoid(gi[:, D:2 * D] + bhh[:, D:2 * D])
    n = jnp.tanh(gi[:, 2 * D:3 * D] + r * bhh[:, 2 * D:3 * D])
    o_ref[...] = jnp.maximum((1.0 - z) * n, 0.0)


def _tc_gcn_post(aggs, dega, degb, b, wih, bih, bhh):
    return pl.pallas_call(
        _tc_gcn_post_body,
        grid=(GRID,),
        in_specs=[_row_spec(D), _row_spec(D), _row_spec(D), _row_spec(D),
                  _full_spec((1, D)), _full_spec((3 * D, D)),
                  _full_spec((1, 3 * D)), _full_spec((1, 3 * D))],
        out_specs=_row_spec(D),
        out_shape=jax.ShapeDtypeStruct((N, D), jnp.float32),
    )(aggs[0], aggs[1], dega, degb, b, wih, bih, bhh)


def _tc_gat_proj_body(f_ref, w_ref, al_ref, ar_ref, feat_ref, el_ref,
                      er_ref, ml_ref, mr_ref):
    feat = lax.dot_general(f_ref[...], w_ref[...], (((1,), (1,)), ((), ())),
                           preferred_element_type=jnp.float32)
    feat_ref[...] = feat
    el = jnp.dot(feat, al_ref[...], preferred_element_type=jnp.float32)
    er = jnp.dot(feat, ar_ref[...], preferred_element_type=jnp.float32)
    el_ref[...] = el
    er_ref[...] = er
    ml = jnp.max(el, axis=0, keepdims=True)
    mr = jnp.max(er, axis=0, keepdims=True)

    @pl.when(pl.program_id(0) == 0)
    def _():
        ml_ref[...] = ml
        mr_ref[...] = mr

    @pl.when(pl.program_id(0) > 0)
    def _():
        ml_ref[...] = jnp.maximum(ml_ref[...], ml)
        mr_ref[...] = jnp.maximum(mr_ref[...], mr)


def _tc_gat_proj(f2, w, al128, ar128):
    return pl.pallas_call(
        _tc_gat_proj_body,
        grid=(GRID,),
        in_specs=[_row_spec(D), _full_spec((HEADS * HDIM, D)),
                  _full_spec((HEADS * HDIM, D)),
                  _full_spec((HEADS * HDIM, D))],
        out_specs=[_row_spec(HEADS * HDIM), _row_spec(D), _row_spec(D),
                   _full_spec((1, D)), _full_spec((1, D))],
        out_shape=[
            jax.ShapeDtypeStruct((N, HEADS * HDIM), jnp.float32),
            jax.ShapeDtypeStruct((N, D), jnp.float32),
            jax.ShapeDtypeStruct((N, D), jnp.float32),
            jax.ShapeDtypeStruct((1, D), jnp.float32),
            jax.ShapeDtypeStruct((1, D), jnp.float32),
        ],
    )(f2, w, al128, ar128)


def _tc_sem_body(u0a_ref, u0b_ref, u1a_ref, u1b_ref, d0a_ref, d0b_ref,
                 d1a_ref, d1b_ref, rep_ref, b0_ref, b1_ref,
                 w1_ref, sb1_ref, w2_ref, h_ref, w1a_ref, w1b_ref, w1c_ref,
                 o1_ref, o2_ref, o3_ref):
    def elu(x):
        return jnp.where(x > 0, x, jnp.exp(jnp.minimum(x, 0.0)) - 1.0)

    def semantic(ua, ub, da, db, b):
        den = jnp.dot(da + db, rep_ref[...],
                      preferred_element_type=jnp.float32)
        den = jnp.where(den > 0, den, 1.0)
        return elu((ua + ub) / den + b)

    sem0 = semantic(u0a_ref[...], u0b_ref[...], d0a_ref[...], d0b_ref[...],
                    b0_ref[...])
    sem1 = semantic(u1a_ref[...], u1b_ref[...], d1a_ref[...], d1b_ref[...],
                    b1_ref[...])

    def score(s):
        t = jnp.tanh(lax.dot_general(s, w1_ref[...], (((1,), (1,)), ((), ())),
                                     preferred_element_type=jnp.float32)
                     + sb1_ref[...])
        return jnp.sum(t * w2_ref[...], axis=1, keepdims=True)

    w0 = score(sem0)
    w1 = score(sem1)
    m = jnp.maximum(w0, w1)
    e0 = jnp.exp(w0 - m)
    e1 = jnp.exp(w1 - m)
    beta0 = e0 / (e0 + e1)
    ui = beta0 * sem0 + (1.0 - beta0) * sem1
    o1_ref[...] = lax.dot_general(h_ref[...], w1a_ref[...],
                                  (((1,), (1,)), ((), ())),
                                  preferred_element_type=jnp.float32)
    o2_ref[...] = lax.dot_general(ui, w1b_ref[...], (((1,), (1,)), ((), ())),
                                  preferred_element_type=jnp.float32)
    o3_ref[...] = lax.dot_general(ui, w1c_ref[...], (((1,), (1,)), ((), ())),
                                  preferred_element_type=jnp.float32)


def _tc_sem(u0a, u0b, u1a, u1b, d0a, d0b, d1a, d1b, rep, b0, b1, saw1, sab1,
            saw2, h, w1a, w1b, w1c):
    hh = HEADS * HDIM
    return pl.pallas_call(
        _tc_sem_body,
        grid=(GRID,),
        in_specs=[_row_spec(hh), _row_spec(hh), _row_spec(hh), _row_spec(hh),
                  _row_spec(D), _row_spec(D), _row_spec(D), _row_spec(D),
                  _full_spec((D, hh)),
                  _full_spec((1, hh)), _full_spec((1, hh)),
                  _full_spec((SA_H, hh)), _full_spec((1, SA_H)),
                  _full_spec((1, SA_H)), _row_spec(D),
                  _full_spec((FC_H, D)), _full_spec((FC_H, hh)),
                  _full_spec((FC_H, hh))],
        out_specs=[_row_spec(FC_H), _row_spec(FC_H), _row_spec(FC_H)],
        out_shape=[jax.ShapeDtypeStruct((N, FC_H), jnp.float32)] * 3,
    )(u0a, u0b, u1a, u1b, d0a, d0b, d1a, d1b, rep, b0, b1, saw1, sab1,
      saw2, h, w1a, w1b, w1c)


def _tc_fc_body(sa_ref, sb_ref, sc_ref, b1_ref, w2_ref, b2_ref, o_ref):
    x = sa_ref[...] + sb_ref[...] + sc_ref[...] + b1_ref[...]
    o_ref[...] = (jnp.sum(x * w2_ref[...], axis=1, keepdims=True)
                  + b2_ref[...])


def _tc_fc(sa, sb, sc, b1, w2, b2):
    rb = 512
    return pl.pallas_call(
        _tc_fc_body,
        grid=(BPAIR // rb,),
        in_specs=[pl.BlockSpec((rb, FC_H), lambda i: (i, 0))] * 3
        + [_full_spec((1, FC_H)), _full_spec((1, FC_H)),
           _full_spec((1, 1))],
        out_specs=pl.BlockSpec((rb, 1), lambda i: (i, 0)),
        out_shape=jax.ShapeDtypeStruct((BPAIR, 1), jnp.float32),
    )(sa, sb, sc, b1, w2, b2)


def kernel(feat1, feat2, edge_index_homo, edge_index_mp0, edge_index_mp1,
           pairs, rgcn_gcn_W, rgcn_gcn_b, rgcn_gru_Wih, rgcn_gru_bih,
           rgcn_gru_bhh, han_gat_W, han_attn_l, han_attn_r, han_gat_b,
           sa_W1, sa_b1, sa_W2, fc_W1, fc_b1, fc_W2, fc_b2):
    src_h = edge_index_homo[0].reshape(NW, CHUNKS, CH)
    dst_h = edge_index_homo[1].reshape(NW, CHUNKS, CH)

    degs = _sc_degrees(src_h, dst_h)
    deg_a, deg_b = degs[0], degs[1]

    h = feat1
    for i in range(NBLK):
        hp = _tc_gcn_pre(h, deg_a, deg_b, rgcn_gcn_W[i])
        aggs = _sc_gcn_agg(hp, src_h, dst_h)
        h = _tc_gcn_post(aggs, deg_a, deg_b, rgcn_gcn_b[i].reshape(1, D),
                         rgcn_gru_Wih[i], rgcn_gru_bih[i].reshape(1, 3 * D),
                         rgcn_gru_bhh[i].reshape(1, 3 * D))
    user_feat = h

    hh = HEADS * HDIM
    eye = jnp.eye(HEADS, D, dtype=jnp.float32)
    gat_u = []
    gat_d = []
    for i, ei in enumerate((edge_index_mp0, edge_index_mp1)):
        src = ei[0].reshape(NW, CHUNKS, CH)
        dst = ei[1].reshape(NW, CHUNKS, CH)
        al = han_attn_l[i]
        ar = han_attn_r[i]
        al128 = (al[:, :, None] * eye[:, None, :]).reshape(hh, D)
        ar128 = (ar[:, :, None] * eye[:, None, :]).reshape(hh, D)
        feat, el, er, ml, mr = _tc_gat_proj(feat2, han_gat_W[i], al128, ar128)
        a_e = _sc_gat_edge(el, er, ml, mr, src, dst)
        den_parts = _sc_gat_den(a_e, dst)
        parts = []
        for s in range(4):
            fs = feat[:, s * D:(s + 1) * D]
            parts.append(_SC_GAT_AGG[s](fs, a_e, src, dst))
        gat_u.append((jnp.concatenate([p[0] for p in parts], axis=1),
                      jnp.concatenate([p[1] for p in parts], axis=1)))
        gat_d.append(den_parts)

    rep = (eye.T[:, :, None] * jnp.ones((1, 1, HDIM))).reshape(D, hh)
    w1a = fc_W1[:, :D]
    w1b = fc_W1[:, D:D + hh]
    w1c = fc_W1[:, D + hh:]
    g1, g2, g3 = _tc_sem(
        gat_u[0][0], gat_u[0][1], gat_u[1][0], gat_u[1][1],
        gat_d[0][0], gat_d[0][1], gat_d[1][0], gat_d[1][1], rep,
        han_gat_b[0].reshape(1, hh), han_gat_b[1].reshape(1, hh),
        sa_W1, sa_b1.reshape(1, SA_H), sa_W2.reshape(1, SA_H),
        user_feat, w1a, w1b, w1c)

    p0 = pairs[:, 0]
    p1 = pairs[:, 1]
    g = _sc_pair_gather(g1, g2, g3, p0, p1)
    out = _tc_fc(g[0], g[1], g[2], fc_b1.reshape(1, FC_H),
                 fc_W2.reshape(1, FC_H), fc_b2.reshape(1, 1))
    return out
